# baseline (speedup 1.0000x reference)
import functools

import jax
import jax.numpy as jnp
from jax import lax
from jax.experimental import pallas as pl
from jax.experimental.pallas import tpu as pltpu
from jax.experimental.pallas import tpu_sc as plsc

_ENTITY_SIZE = 1000000
_EMBED_DIM = 64
_MARGIN = 1.0

_L = 16
_NQ = _EMBED_DIM // _L
_CHUNK = 128
_SLOT = 2 * _L
_PAIR = 2 * _EMBED_DIM


def _rsqrt_nr(x):
    i = lax.bitcast_convert_type(x, jnp.int32)
    y = lax.bitcast_convert_type(jnp.int32(0x5F3759DF) - (i >> 1), jnp.float32)
    for _ in range(3):
        y = y * (1.5 - 0.5 * x * y * y)
    return y


def _lane_sum_splat(q, scr, s0):
    scr[pl.ds(s0, _L)] = q
    scr[pl.ds(s0 + _L, _L)] = q
    s = q + scr[pl.ds(s0 + 4, _L)] + scr[pl.ds(s0 + 8, _L)] \
        + scr[pl.ds(s0 + 12, _L)]
    scr[pl.ds(s0, _L)] = s
    scr[pl.ds(s0 + _L, _L)] = s
    return s + scr[pl.ds(s0 + 1, _L)] + scr[pl.ds(s0 + 2, _L)] \
        + scr[pl.ds(s0 + 3, _L)]


def _triplet_distance(hrow_v, rrow_v, trow_v, offs, eh_g, et_g, scr, i, j):
    ho, ro, to = offs
    h = [hrow_v[i, pl.ds(ho + k * _L, _L)] for k in range(_NQ)]
    r = [rrow_v[i, pl.ds(ro + k * _L, _L)] for k in range(_NQ)]
    t = [trow_v[i, pl.ds(to + k * _L, _L)] for k in range(_NQ)]

    hh_p = h[0] * h[0]
    tt_p = t[0] * t[0]
    for k in range(1, _NQ):
        hh_p = hh_p + h[k] * h[k]
        tt_p = tt_p + t[k] * t[k]
    hh = _lane_sum_splat(hh_p, scr, (2 * j) * _SLOT)
    tt = _lane_sum_splat(tt_p, scr, (2 * j + 1) * _SLOT)

    eh = jnp.full((_L,), 0.0, jnp.float32) + eh_g[j]
    et = jnp.full((_L,), 0.0, jnp.float32) + et_g[j]
    a = _rsqrt_nr(hh)
    a = a + eh * (1.0 - a)
    b = _rsqrt_nr(tt)
    b = b + et * (1.0 - b)

    ss_p = jnp.zeros((_L,), jnp.float32)
    for k in range(_NQ):
        s = h[k] * a + r[k] - t[k] * b
        ss_p = ss_p + s * s
    d2 = _lane_sum_splat(ss_p, scr, (2 * j) * _SLOT)
    return jnp.where(d2 > 0.0, d2 * _rsqrt_nr(d2),
                     jnp.zeros((_L,), jnp.float32))


def _transe_sc(hidx_all, ridx_all, tidx_all, ent2, rel2, batch):
    info = plsc.get_sparse_core_info()
    nw = info.num_cores * info.num_subcores
    per_w = batch // nw
    n_chunks = per_w // _CHUNK
    mesh = plsc.VectorSubcoreMesh(core_axis_name="c", subcore_axis_name="s")

    @functools.partial(
        pl.kernel,
        mesh=mesh,
        out_type=jax.ShapeDtypeStruct((batch,), jnp.float32),
        scratch_types=[
            pltpu.VMEM((_CHUNK,), jnp.int32),
            pltpu.VMEM((_CHUNK,), jnp.int32),
            pltpu.VMEM((_CHUNK,), jnp.int32),
            pltpu.VMEM((_CHUNK,), jnp.int32),
            pltpu.VMEM((_CHUNK,), jnp.int32),
            pltpu.VMEM((_CHUNK,), jnp.int32),
            pltpu.VMEM((_CHUNK, _PAIR), jnp.float32),
            pltpu.VMEM((_CHUNK, _PAIR), jnp.float32),
            pltpu.VMEM((_CHUNK, _PAIR), jnp.float32),
            pltpu.VMEM((2 * _L * _SLOT,), jnp.float32),
            pltpu.VMEM((2 * per_w,), jnp.float32),
            pltpu.VMEM((per_w,), jnp.float32),
            pltpu.SemaphoreType.DMA,
        ],
    )
    def k(hidx_h, ridx_h, tidx_h, ent_h, rel_h, out_h,
          hidx_v, ridx_v, tidx_v, hg_v, rg_v, tg_v,
          hrow_v, rrow_v, trow_v, scr_v, dist_v, loss_v, sem):
        wid = lax.axis_index("s") * info.num_cores + lax.axis_index("c")
        wbase = wid * per_w
        lanes = lax.iota(jnp.int32, _L)

        def chunk_body(c, carry):
            p = c // n_chunks
            cc = c - p * n_chunks
            src = p * batch + wbase + cc * _CHUNK
            pltpu.sync_copy(hidx_h.at[pl.ds(src, _CHUNK)], hidx_v)
            pltpu.sync_copy(ridx_h.at[pl.ds(src, _CHUNK)], ridx_v)
            pltpu.sync_copy(tidx_h.at[pl.ds(src, _CHUNK)], tidx_v)

            def halve_body(g, carry2):
                gb = g * _L
                hg_v[pl.ds(gb, _L)] = hidx_v[pl.ds(gb, _L)] >> 1
                rg_v[pl.ds(gb, _L)] = ridx_v[pl.ds(gb, _L)] >> 1
                tg_v[pl.ds(gb, _L)] = tidx_v[pl.ds(gb, _L)] >> 1
                return carry2

            lax.fori_loop(0, _CHUNK // _L, halve_body, 0)

            cp_h = pltpu.async_copy(ent_h.at[hg_v], hrow_v, sem)
            cp_r = pltpu.async_copy(rel_h.at[rg_v], rrow_v, sem)
            cp_t = pltpu.async_copy(ent_h.at[tg_v], trow_v, sem)
            cp_h.wait()
            cp_r.wait()
            cp_t.wait()

            dbase = p * per_w + cc * _CHUNK

            def group_body(g, carry2):
                gb = g * _L
                last = jnp.full((_L,), _ENTITY_SIZE - 1, jnp.int32)
                onef = jnp.ones((_L,), jnp.float32)
                zerof = jnp.zeros((_L,), jnp.float32)
                hidx_g = hidx_v[pl.ds(gb, _L)]
                ridx_g = ridx_v[pl.ds(gb, _L)]
                tidx_g = tidx_v[pl.ds(gb, _L)]
                eh_g = jnp.where(hidx_g == last, onef, zerof)
                et_g = jnp.where(tidx_g == last, onef, zerof)
                hpar = (hidx_g & 1) * _EMBED_DIM
                rpar = (ridx_g & 1) * _EMBED_DIM
                tpar = (tidx_g & 1) * _EMBED_DIM
                d_acc = jnp.zeros((_L,), jnp.float32)
                for j in range(_L):
                    offs = (hpar[j], rpar[j], tpar[j])
                    d = _triplet_distance(hrow_v, rrow_v, trow_v, offs,
                                          eh_g, et_g, scr_v, gb + j, j)
                    d_acc = jnp.where(lanes == j, d, d_acc)
                dist_v[pl.ds(dbase + gb, _L)] = d_acc
                return carry2

            lax.fori_loop(0, _CHUNK // _L, group_body, 0)
            return carry

        lax.fori_loop(0, 2 * n_chunks, chunk_body, 0)

        def loss_body(g, carry):
            gb = g * _L
            dp = dist_v[pl.ds(gb, _L)]
            dn = dist_v[pl.ds(per_w + gb, _L)]
            loss_v[pl.ds(gb, _L)] = jnp.maximum(dp - dn + _MARGIN, 0.0)
            return carry

        lax.fori_loop(0, per_w // _L, loss_body, 0)
        pltpu.sync_copy(loss_v, out_h.at[pl.ds(wbase, per_w)])

    return k(hidx_all, ridx_all, tidx_all, ent2, rel2)


def kernel(positive_triplets, negative_triplets, entity_emb, relation_emb):
    batch = positive_triplets.shape[0]
    cols = jnp.concatenate(
        [positive_triplets.astype(jnp.int32),
         negative_triplets.astype(jnp.int32)], axis=0).T
    ent2 = entity_emb.reshape(_ENTITY_SIZE // 2, _PAIR)
    rel2 = relation_emb.reshape(relation_emb.shape[0] // 2, _PAIR)
    return _transe_sc(cols[0], cols[1], cols[2], ent2, rel2, batch)


# speedup vs baseline: 2.1532x; 2.1532x over previous
import functools

import jax
import jax.numpy as jnp
from jax import lax
from jax.experimental import pallas as pl
from jax.experimental.pallas import tpu as pltpu
from jax.experimental.pallas import tpu_sc as plsc

_ENTITY_SIZE = 1000000
_EMBED_DIM = 64
_MARGIN = 1.0

_L = 16
_CHUNK = 128
_NG = _CHUNK // _L

_EPAD = _ENTITY_SIZE + 64
_NTILE = _EPAD // 128
_TROW = _NTILE * 1024
_FLAT = _EMBED_DIM * _EPAD


def _rsqrt_nr(x):
    i = lax.bitcast_convert_type(x, jnp.int32)
    y = lax.bitcast_convert_type(jnp.int32(0x5F3759DF) - (i >> 1), jnp.float32)
    for _ in range(3):
        y = y * (1.5 - 0.5 * x * y * y)
    return y


def _phys_view(table):
    return (jnp.pad(table, ((0, _EPAD - table.shape[0]), (0, 0))).T
            .reshape(8, 8, _NTILE, 128)
            .transpose(0, 2, 1, 3)
            .reshape(_FLAT))


def _transe_sc(hidx_all, ridx_all, tidx_all, ent_lin, rel_lin, batch):
    info = plsc.get_sparse_core_info()
    nw = info.num_cores * info.num_subcores
    per_w = batch // nw
    n_chunks = per_w // _CHUNK
    mesh = plsc.VectorSubcoreMesh(core_axis_name="c", subcore_axis_name="s")

    @functools.partial(
        pl.kernel,
        mesh=mesh,
        out_type=jax.ShapeDtypeStruct((batch,), jnp.float32),
        scratch_types=[
            pltpu.VMEM((_CHUNK,), jnp.int32),
            pltpu.VMEM((_CHUNK,), jnp.int32),
            pltpu.VMEM((_CHUNK,), jnp.int32),
            pltpu.VMEM((_EMBED_DIM, _CHUNK), jnp.int32),
            pltpu.VMEM((_EMBED_DIM, _CHUNK), jnp.int32),
            pltpu.VMEM((_EMBED_DIM, _CHUNK), jnp.int32),
            pltpu.VMEM((_EMBED_DIM, _CHUNK), jnp.float32),
            pltpu.VMEM((_EMBED_DIM, _CHUNK), jnp.float32),
            pltpu.VMEM((_EMBED_DIM, _CHUNK), jnp.float32),
            pltpu.VMEM((2 * per_w,), jnp.float32),
            pltpu.VMEM((per_w,), jnp.float32),
            pltpu.SemaphoreType.DMA,
        ],
    )
    def k(hidx_h, ridx_h, tidx_h, ent_h, rel_h, out_h,
          hidx_v, ridx_v, tidx_v, hgi_v, rgi_v, tgi_v,
          hval_v, rval_v, tval_v, dist_v, loss_v, sem):
        wid = lax.axis_index("s") * info.num_cores + lax.axis_index("c")
        wbase = wid * per_w

        def chunk_body(c, carry):
            p = c // n_chunks
            cc = c - p * n_chunks
            src = p * batch + wbase + cc * _CHUNK
            pltpu.sync_copy(hidx_h.at[pl.ds(src, _CHUNK)], hidx_v)
            pltpu.sync_copy(ridx_h.at[pl.ds(src, _CHUNK)], ridx_v)
            pltpu.sync_copy(tidx_h.at[pl.ds(src, _CHUNK)], tidx_v)

            def genf_body(f, carry2):
                fc = (f >> 3) * _TROW + (f & 7) * 128

                def geng_body(g, carry3):
                    gb = g * _L
                    for src_v, dst_v in ((hidx_v, hgi_v), (ridx_v, rgi_v),
                                         (tidx_v, tgi_v)):
                        e = src_v[pl.ds(gb, _L)]
                        base = ((e >> 7) << 10) + (e & 127)
                        dst_v[f, pl.ds(gb, _L)] = base + fc
                    return carry3

                lax.fori_loop(0, _NG, geng_body, 0)
                pltpu.async_copy(ent_h.at[hgi_v.at[f]], hval_v.at[f], sem)
                pltpu.async_copy(rel_h.at[rgi_v.at[f]], rval_v.at[f], sem)
                pltpu.async_copy(ent_h.at[tgi_v.at[f]], tval_v.at[f], sem)
                return carry2

            lax.fori_loop(0, _EMBED_DIM, genf_body, 0)

            def drain_body(f, carry2):
                pltpu.make_async_copy(ent_h.at[hgi_v.at[f]], hval_v.at[f],
                                      sem).wait()
                pltpu.make_async_copy(rel_h.at[rgi_v.at[f]], rval_v.at[f],
                                      sem).wait()
                pltpu.make_async_copy(ent_h.at[tgi_v.at[f]], tval_v.at[f],
                                      sem).wait()
                return carry2

            lax.fori_loop(0, _EMBED_DIM, drain_body, 0)

            dbase = p * per_w + cc * _CHUNK

            def group_body(g, carry2):
                gb = g * _L
                zl = jnp.zeros((_L,), jnp.float32)

                def acc_body(f, accs):
                    hh, tt, rr, hr, ht, rt = accs
                    h = hval_v[f, pl.ds(gb, _L)]
                    r = rval_v[f, pl.ds(gb, _L)]
                    t = tval_v[f, pl.ds(gb, _L)]
                    return (hh + h * h, tt + t * t, rr + r * r,
                            hr + h * r, ht + h * t, rt + r * t)

                hh, tt, rr, hr, ht, rt = lax.fori_loop(
                    0, _EMBED_DIM, acc_body, (zl, zl, zl, zl, zl, zl))

                last = jnp.full((_L,), _ENTITY_SIZE - 1, jnp.int32)
                onef = jnp.ones((_L,), jnp.float32)
                eh = jnp.where(hidx_v[pl.ds(gb, _L)] == last, onef, zl)
                et = jnp.where(tidx_v[pl.ds(gb, _L)] == last, onef, zl)
                a = _rsqrt_nr(hh)
                a = a + eh * (1.0 - a)
                b = _rsqrt_nr(tt)
                b = b + et * (1.0 - b)
                d2 = hh * (a * a) + rr + tt * (b * b) \
                    + 2.0 * (a * hr - (a * b) * ht - b * rt)
                d2 = jnp.maximum(d2, 0.0)
                d = jnp.where(d2 > 0.0, d2 * _rsqrt_nr(d2), zl)
                dist_v[pl.ds(dbase + gb, _L)] = d
                return carry2

            lax.fori_loop(0, _NG, group_body, 0)
            return carry

        lax.fori_loop(0, 2 * n_chunks, chunk_body, 0)

        def loss_body(g, carry):
            gb = g * _L
            dp = dist_v[pl.ds(gb, _L)]
            dn = dist_v[pl.ds(per_w + gb, _L)]
            loss_v[pl.ds(gb, _L)] = jnp.maximum(dp - dn + _MARGIN, 0.0)
            return carry

        lax.fori_loop(0, per_w // _L, loss_body, 0)
        pltpu.sync_copy(loss_v, out_h.at[pl.ds(wbase, per_w)])

    return k(hidx_all, ridx_all, tidx_all, ent_lin, rel_lin)


def kernel(positive_triplets, negative_triplets, entity_emb, relation_emb):
    batch = positive_triplets.shape[0]
    cols = jnp.concatenate(
        [positive_triplets.astype(jnp.int32),
         negative_triplets.astype(jnp.int32)], axis=0).T
    return _transe_sc(cols[0], cols[1], cols[2],
                      _phys_view(entity_emb), _phys_view(relation_emb), batch)
